# baseline (speedup 1.0000x reference)
import functools

import jax
import jax.numpy as jnp
from jax import lax
from jax.experimental import pallas as pl
from jax.experimental.pallas import tpu as pltpu
from jax.experimental.pallas import tpu_sc as plsc

B, H, W, C = 16, 384, 384, 3
N = B * H * W
NTILES = 32
PER_TILE = N // NTILES
SUB = 8192
NSUB = PER_TILE // SUB
STEP = 128
NSTEP = SUB // STEP


def _sampler_body(coords_hbm, imgs_hbm, out_hbm, cbuf,
                  ib00, ib01, ib10, ib11, gb00, gb01, gb10, gb11,
                  fbx, fby, obuf, sem):
    wid = lax.axis_index("s") * 2 + lax.axis_index("c")
    ebase = wid * PER_TILE
    bflat = (wid // 2) * (H * W)

    def subchunk(s, carry):
        e0 = pl.multiple_of(ebase + s * SUB, SUB)
        pltpu.sync_copy(coords_hbm.at[pl.ds(e0 * 2, 2 * SUB)], cbuf)

        def compute_fire(j, carry2):
            iota = lax.iota(jnp.int32, 16)
            for i2 in range(STEP // 16):
                off = j * (2 * STEP) + i2 * 32
                xsel = off + iota * 2
                xv = plsc.load_gather(cbuf, [xsel])
                yv = plsc.load_gather(cbuf, [xsel + 1])
                x0 = xv.astype(jnp.int32)
                y0 = yv.astype(jnp.int32)
                fx = xv - x0.astype(jnp.float32)
                fy = yv - y0.astype(jnp.float32)
                x0c = jnp.minimum(x0, W - 1)
                x1c = jnp.minimum(x0 + 1, W - 1)
                r0 = bflat + jnp.minimum(y0, H - 1) * W
                r1 = bflat + jnp.minimum(y0 + 1, H - 1) * W
                c = i2 * 16
                ib00[j, pl.ds(c, 16)] = r0 + x0c
                ib01[j, pl.ds(c, 16)] = r1 + x0c
                ib10[j, pl.ds(c, 16)] = r0 + x1c
                ib11[j, pl.ds(c, 16)] = r1 + x1c
                fbx[pl.ds(j * STEP + c, 16)] = fx
                fby[pl.ds(j * STEP + c, 16)] = fy
            pltpu.async_copy(imgs_hbm.at[ib00.at[j]], gb00.at[j], sem)
            pltpu.async_copy(imgs_hbm.at[ib01.at[j]], gb01.at[j], sem)
            pltpu.async_copy(imgs_hbm.at[ib10.at[j]], gb10.at[j], sem)
            pltpu.async_copy(imgs_hbm.at[ib11.at[j]], gb11.at[j], sem)
            return carry2

        lax.fori_loop(0, NSTEP, compute_fire, 0)

        def drain(j, carry2):
            pltpu.make_async_copy(imgs_hbm.at[ib00.at[j]], gb00.at[j], sem).wait()
            pltpu.make_async_copy(imgs_hbm.at[ib01.at[j]], gb01.at[j], sem).wait()
            pltpu.make_async_copy(imgs_hbm.at[ib10.at[j]], gb10.at[j], sem).wait()
            pltpu.make_async_copy(imgs_hbm.at[ib11.at[j]], gb11.at[j], sem).wait()
            return carry2

        lax.fori_loop(0, NSTEP, drain, 0)

        def combine(j, carry2):
            for i2 in range(STEP // 16):
                c = i2 * 16
                g00 = gb00[j, pl.ds(c, 16)]
                g01 = gb01[j, pl.ds(c, 16)]
                g10 = gb10[j, pl.ds(c, 16)]
                g11 = gb11[j, pl.ds(c, 16)]
                fx = fbx[pl.ds(j * STEP + c, 16)]
                fy = fby[pl.ds(j * STEP + c, 16)]
                wx0 = 1.0 - fx
                wy0 = 1.0 - fy
                res = (wx0 * wy0) * g00 + (wx0 * fy) * g01
                res = res + ((fx * wy0) * g10 + (fx * fy) * g11)
                obuf[pl.ds(j * STEP + c, 16)] = res
            return carry2

        lax.fori_loop(0, NSTEP, combine, 0)
        pltpu.sync_copy(obuf, out_hbm.at[pl.ds(e0, SUB)])
        return carry

    lax.fori_loop(0, NSUB, subchunk, 0)


def kernel(imgs, coords):
    flat = imgs.reshape(-1)
    cflat = coords.reshape(-1)
    mesh = plsc.VectorSubcoreMesh(core_axis_name="c", subcore_axis_name="s")
    run = functools.partial(
        pl.kernel,
        mesh=mesh,
        compiler_params=pltpu.CompilerParams(needs_layout_passes=False),
        out_type=jax.ShapeDtypeStruct((N,), jnp.float32),
        scratch_types=[
            pltpu.VMEM((2 * SUB,), jnp.float32),
            pltpu.VMEM((NSTEP, STEP), jnp.int32),
            pltpu.VMEM((NSTEP, STEP), jnp.int32),
            pltpu.VMEM((NSTEP, STEP), jnp.int32),
            pltpu.VMEM((NSTEP, STEP), jnp.int32),
            pltpu.VMEM((NSTEP, STEP), jnp.float32),
            pltpu.VMEM((NSTEP, STEP), jnp.float32),
            pltpu.VMEM((NSTEP, STEP), jnp.float32),
            pltpu.VMEM((NSTEP, STEP), jnp.float32),
            pltpu.VMEM((SUB,), jnp.float32),
            pltpu.VMEM((SUB,), jnp.float32),
            pltpu.VMEM((SUB,), jnp.float32),
            pltpu.SemaphoreType.DMA,
        ],
    )(_sampler_body)
    out = run(cflat, flat)
    return out.reshape(B, H, W, 1)
